# baseline (speedup 1.0000x reference)
import jax
import jax.numpy as jnp
from jax.experimental import pallas as pl
from jax.experimental.pallas import tpu as pltpu

DIM = 128
HEADS = 4
HD = DIM // HEADS
HID = 512
BN_EPS = 1e-5
LN_EPS = 1e-5
NEG = -1e30
LOG2E = 1.4426950408889634
EXP2_OFF = -64.0


def _layernorm(x, g, b):
    m = jnp.mean(x, axis=-1, keepdims=True)
    v = jnp.mean((x - m) ** 2, axis=-1, keepdims=True)
    return (x - m) * jax.lax.rsqrt(v + LN_EPS) * g + b


def _erf(x):
    a1, a2, a3, a4, a5 = (0.254829592, -0.284496736, 1.421413741,
                          -1.453152027, 1.061405429)
    s = jnp.sign(x)
    ax = jnp.abs(x)
    t = 1.0 / (1.0 + 0.3275911 * ax)
    poly = ((((a5 * t + a4) * t + a3) * t + a2) * t + a1) * t
    y = 1.0 - poly * jnp.exp(-ax * ax)
    return s * y


def _gelu(x):
    return 0.5 * x * (1.0 + _erf(x * 0.7071067811865476))


def _select_body(ks_ref, score_ref, sel_ref):
    k = ks_ref[0, 0, 0]
    si = jax.lax.bitcast_convert_type(score_ref[0], jnp.int32)

    def bit_step(i, t):
        cand = t | (jnp.int32(1) << (jnp.int32(30) - i))
        cnt = jnp.sum((si >= cand).astype(jnp.int32))
        return jnp.where(cnt >= k, cand, t)

    thr = jax.lax.fori_loop(0, 31, bit_step, jnp.int32(0))
    gt = si > thr
    ties = (si == thr).astype(jnp.int32)
    c1 = jnp.sum(gt.astype(jnp.int32))
    need = k - c1
    acc = ties
    np_ = acc.shape[1]
    sh = 1
    while sh < np_:
        z = jnp.zeros((1, sh), jnp.int32)
        acc = acc + jnp.concatenate([z, acc[:, : np_ - sh]], axis=1)
        sh *= 2
    rank_excl = acc - ties
    sel = gt | ((ties > 0) & (rank_excl < need))
    sel_ref[0] = sel.astype(jnp.float32)


def _mixer_body(hs_ref, xall_ref, xq_ref, selc_ref, selq_ref, wc_ref,
                lng_ref, lnb_ref, inwt_ref, inb_ref, outwt_ref, outb_ref,
                f1t_ref, f1b_ref, f2t_ref, f2b_ref,
                out_ref, qb_ref, kvb_ref):
    q_idx = pl.program_id(1)
    BQ = out_ref.shape[1]
    scale2 = LOG2E / (HD ** 0.5)

    @pl.when(q_idx == 0)
    def _():
        for m in range(2):
            x_all = xall_ref[m, 0]
            xn = _layernorm(x_all, lng_ref[m], lnb_ref[m])
            qkv = jnp.dot(xn, inwt_ref[m],
                          preferred_element_type=jnp.float32) + inb_ref[m]
            qb_ref[m] = (qkv[:, :DIM] * scale2).astype(jnp.bfloat16)
            kvb_ref[m] = qkv[:, DIM:].astype(jnp.bfloat16)

    sel = selc_ref[0]
    bias = jnp.where(sel > 0.0, EXP2_OFF, NEG)

    rs = []
    for m in range(2):
        xq = xq_ref[m, 0]
        outs = []
        for h in range(HEADS):
            qh = qb_ref[m, pl.ds(q_idx * BQ, BQ), h * HD:(h + 1) * HD]
            kh = kvb_ref[m, :, h * HD:(h + 1) * HD]
            vh = kvb_ref[m, :, DIM + h * HD: DIM + (h + 1) * HD]
            s = jax.lax.dot_general(
                qh, kh, (((1,), (1,)), ((), ())),
                preferred_element_type=jnp.float32) + bias
            p = jnp.exp2(s)
            l = jnp.sum(p, axis=1, keepdims=True)
            oh = jnp.dot(p.astype(jnp.bfloat16), vh,
                         preferred_element_type=jnp.float32) / l
            outs.append(oh)
        o = jnp.concatenate(outs, axis=1)
        att = jnp.dot(o, outwt_ref[m], preferred_element_type=jnp.float32)
        x1 = xq + att + outb_ref[m]
        h1 = _layernorm(x1, lng_ref[m], lnb_ref[m])
        h1 = jnp.dot(h1, f1t_ref[m], preferred_element_type=jnp.float32)
        h1 = _gelu(h1 + f1b_ref[m])
        h2 = jnp.dot(h1, f2t_ref[m], preferred_element_type=jnp.float32)
        rs.append(x1 + h2 + f2b_ref[m])

    hscale = hs_ref[0, 0]
    w = jnp.broadcast_to(wc_ref[0], (BQ, DIM))
    selq = selq_ref[0]
    base = xq_ref[0, 0] + xq_ref[1, 0]
    score = jnp.abs(w - 0.5)
    delta = rs[0] * w + rs[1] * (1.0 - w)
    mod = 1.0 + score * hscale
    edge = (base + delta) * mod
    out_ref[0] = jnp.where(selq > 0.0, edge, base)


def kernel(f_ir, f_vis, params):
    B, C, H, W = f_ir.shape
    N = H * W
    NP = ((N + 127) // 128) * 128
    p = params

    x2 = jnp.concatenate([f_ir, f_vis], axis=1)
    a = jnp.einsum('oc,bchw->bohw', p['agent_w1'], x2) \
        + p['agent_b1'][None, :, None, None]
    a = a / jnp.sqrt(1.0 + BN_EPS) * p['agent_bn_g'][None, :, None, None] \
        + p['agent_bn_b'][None, :, None, None]
    a = jax.nn.relu(a)
    a = jnp.einsum('oc,bchw->bohw', p['agent_w2'], a) \
        + p['agent_b2'][None, :, None, None]
    weights = jax.nn.sigmoid(a)

    imp = jnp.abs(weights - 0.5)
    gs = imp.mean(axis=(2, 3)).reshape(-1, 1)
    hh = jax.nn.relu(gs @ p['hyp_w1'].T + p['hyp_b1'])
    kr = jax.nn.sigmoid(hh @ p['hyp_w2'].T + p['hyp_b2']) * 0.8 + 0.1
    krv = kr[:, 0]
    c = krv * jnp.float32(4097.0)
    hi = c - (c - krv)
    lo = krv - hi
    prod_hi = hi * jnp.float32(N)
    prod_lo = lo * jnp.float32(N)
    fh = jnp.floor(prod_hi)
    frac = (prod_hi - fh) + prod_lo
    kf = fh + jnp.floor(frac)
    ks = jnp.maximum(kf.astype(jnp.int32), 64).reshape(B, 1, 1)

    score_flat = imp.reshape(B, 1, N)
    score_row = jnp.pad(score_flat, ((0, 0), (0, 0), (0, NP - N)),
                        constant_values=-1.0)
    w_col = jnp.pad(weights.reshape(B, N, 1),
                    ((0, 0), (0, NP - N), (0, 0)))

    xt_ir = f_ir.reshape(B, C, N).transpose(0, 2, 1)
    xt_vis = f_vis.reshape(B, C, N).transpose(0, 2, 1)
    pad = ((0, 0), (0, NP - N), (0, 0))
    xt_ir = jnp.pad(xt_ir, pad)
    xt_vis = jnp.pad(xt_vis, pad)
    xs = jnp.stack([xt_ir, xt_vis], axis=0)

    sel_row = pl.pallas_call(
        _select_body,
        grid=(B,),
        in_specs=[
            pl.BlockSpec((1, 1, 1), lambda b: (b, 0, 0),
                         memory_space=pltpu.SMEM),
            pl.BlockSpec((1, 1, NP), lambda b: (b, 0, 0)),
        ],
        out_specs=pl.BlockSpec((1, 1, NP), lambda b: (b, 0, 0)),
        out_shape=jax.ShapeDtypeStruct((B, 1, NP), jnp.float32),
    )(ks, score_row)
    sel_col = sel_row.transpose(0, 2, 1)

    def stk(name):
        return jnp.stack([p['mir' + name], p['mvis' + name]], axis=0)

    lng = stk('_ln_g').reshape(2, 1, C)
    lnb = stk('_ln_b').reshape(2, 1, C)
    inwt = jnp.stack([p['mir_in_w'].T, p['mvis_in_w'].T], axis=0)
    inb = stk('_in_b').reshape(2, 1, 3 * C)
    outwt = jnp.stack([p['mir_out_w'].T, p['mvis_out_w'].T], axis=0)
    outb = stk('_out_b').reshape(2, 1, C)
    f1t = jnp.stack([p['mir_f1_w'].T, p['mvis_f1_w'].T], axis=0)
    f1b = stk('_f1_b').reshape(2, 1, 4 * C)
    f2t = jnp.stack([p['mir_f2_w'].T, p['mvis_f2_w'].T], axis=0)
    f2b = stk('_f2_b').reshape(2, 1, C)
    hscale = p['highlight_scale'].reshape(1, 1)

    BQ = 800
    nqb = NP // BQ
    full = lambda b, q: (0, 0)
    fullm = lambda b, q: (0, 0, 0)
    out_tok = pl.pallas_call(
        _mixer_body,
        grid=(B, nqb),
        in_specs=[
            pl.BlockSpec((1, 1), full, memory_space=pltpu.SMEM),
            pl.BlockSpec((2, 1, NP, C), lambda b, q: (0, b, 0, 0)),
            pl.BlockSpec((2, 1, BQ, C), lambda b, q: (0, b, q, 0)),
            pl.BlockSpec((1, 1, NP), lambda b, q: (b, 0, 0)),
            pl.BlockSpec((1, BQ, 1), lambda b, q: (b, q, 0)),
            pl.BlockSpec((1, BQ, 1), lambda b, q: (b, q, 0)),
            pl.BlockSpec((2, 1, C), fullm),
            pl.BlockSpec((2, 1, C), fullm),
            pl.BlockSpec((2, C, 3 * C), fullm),
            pl.BlockSpec((2, 1, 3 * C), fullm),
            pl.BlockSpec((2, C, C), fullm),
            pl.BlockSpec((2, 1, C), fullm),
            pl.BlockSpec((2, C, 4 * C), fullm),
            pl.BlockSpec((2, 1, 4 * C), fullm),
            pl.BlockSpec((2, 4 * C, C), fullm),
            pl.BlockSpec((2, 1, C), fullm),
        ],
        out_specs=pl.BlockSpec((1, BQ, C), lambda b, q: (b, q, 0)),
        out_shape=jax.ShapeDtypeStruct((B, NP, C), jnp.float32),
        scratch_shapes=[
            pltpu.VMEM((2, NP, DIM), jnp.bfloat16),
            pltpu.VMEM((2, NP, 2 * DIM), jnp.bfloat16),
        ],
    )(hscale, xs, xs, sel_row, sel_col, w_col,
      lng, lnb, inwt, inb, outwt, outb, f1t, f1b, f2t, f2b)

    f_final = out_tok[:, :N, :].transpose(0, 2, 1).reshape(B, C, H, W)
    aux_loss = jnp.asarray(0.0, jnp.float32)
    return (f_final, aux_loss)


# speedup vs baseline: 1.0680x; 1.0680x over previous
import jax
import jax.numpy as jnp
from jax.experimental import pallas as pl
from jax.experimental.pallas import tpu as pltpu

DIM = 128
HEADS = 4
HD = DIM // HEADS
HID = 512
BN_EPS = 1e-5
LN_EPS = 1e-5
NEG = -1e30
LOG2E = 1.4426950408889634
EXP2_OFF = -64.0


def _layernorm(x, g, b):
    m = jnp.mean(x, axis=-1, keepdims=True)
    v = jnp.mean((x - m) ** 2, axis=-1, keepdims=True)
    return (x - m) * jax.lax.rsqrt(v + LN_EPS) * g + b


def _erf(x):
    a1, a2, a3, a4, a5 = (0.254829592, -0.284496736, 1.421413741,
                          -1.453152027, 1.061405429)
    s = jnp.sign(x)
    ax = jnp.abs(x)
    t = 1.0 / (1.0 + 0.3275911 * ax)
    poly = ((((a5 * t + a4) * t + a3) * t + a2) * t + a1) * t
    y = 1.0 - poly * jnp.exp(-ax * ax)
    return s * y


def _gelu(x):
    return 0.5 * x * (1.0 + _erf(x * 0.7071067811865476))


def _select_body(ks_ref, score_ref, sel_ref):
    k = ks_ref[0, 0, 0]
    si = jax.lax.bitcast_convert_type(score_ref[0], jnp.int32)

    def bit_step(i, t):
        cand = t | (jnp.int32(1) << (jnp.int32(30) - i))
        cnt = jnp.sum((si >= cand).astype(jnp.int32))
        return jnp.where(cnt >= k, cand, t)

    thr = jax.lax.fori_loop(0, 31, bit_step, jnp.int32(0))
    gt = si > thr
    ties = (si == thr).astype(jnp.int32)
    c1 = jnp.sum(gt.astype(jnp.int32))
    need = k - c1
    acc = ties
    np_ = acc.shape[1]
    sh = 1
    while sh < np_:
        z = jnp.zeros((1, sh), jnp.int32)
        acc = acc + jnp.concatenate([z, acc[:, : np_ - sh]], axis=1)
        sh *= 2
    rank_excl = acc - ties
    sel = gt | ((ties > 0) & (rank_excl < need))
    sel_ref[0] = sel.astype(jnp.float32)


def _mixer_body(hs_ref, xall_ref, xq_ref, selc_ref, selq_ref, wc_ref,
                lng_ref, lnb_ref, inwt_ref, inb_ref, outwt_ref, outb_ref,
                f1t_ref, f1b_ref, f2t_ref, f2b_ref,
                out_ref, qb_ref, kvb_ref):
    q_idx = pl.program_id(1)
    BQ = out_ref.shape[1]
    scale2 = LOG2E / (HD ** 0.5)

    @pl.when(q_idx == 0)
    def _():
        for m in range(2):
            x_all = xall_ref[m, 0]
            xn = _layernorm(x_all, lng_ref[m], lnb_ref[m])
            qkv = jnp.dot(xn, inwt_ref[m],
                          preferred_element_type=jnp.float32) + inb_ref[m]
            qb_ref[m] = (qkv[:, :DIM] * scale2).astype(jnp.bfloat16)
            kvb_ref[m] = qkv[:, DIM:].astype(jnp.bfloat16)

    sel = selc_ref[0]
    bias = jnp.where(sel > 0.0, EXP2_OFF, NEG)

    rs = []
    for m in range(2):
        xq = xq_ref[m, 0]
        outs = []
        for h in range(HEADS):
            qh = qb_ref[m, pl.ds(q_idx * BQ, BQ), h * HD:(h + 1) * HD]
            kh = kvb_ref[m, :, h * HD:(h + 1) * HD]
            vh = kvb_ref[m, :, DIM + h * HD: DIM + (h + 1) * HD]
            s = jax.lax.dot_general(
                qh, kh, (((1,), (1,)), ((), ())),
                preferred_element_type=jnp.float32) + bias
            p = jnp.exp2(s)
            l = jnp.sum(p, axis=1, keepdims=True)
            oh = jnp.dot(p.astype(jnp.bfloat16), vh,
                         preferred_element_type=jnp.float32) / l
            outs.append(oh)
        o = jnp.concatenate(outs, axis=1)
        att = jnp.dot(o, outwt_ref[m], preferred_element_type=jnp.float32)
        x1 = xq + att + outb_ref[m]
        h1 = _layernorm(x1, lng_ref[m], lnb_ref[m])
        h1 = jnp.dot(h1, f1t_ref[m], preferred_element_type=jnp.float32)
        h1 = _gelu(h1 + f1b_ref[m])
        h2 = jnp.dot(h1, f2t_ref[m], preferred_element_type=jnp.float32)
        rs.append(x1 + h2 + f2b_ref[m])

    hscale = hs_ref[0, 0]
    w = jnp.broadcast_to(wc_ref[0], (BQ, DIM))
    selq = selq_ref[0]
    base = xq_ref[0, 0] + xq_ref[1, 0]
    score = jnp.abs(w - 0.5)
    delta = rs[0] * w + rs[1] * (1.0 - w)
    mod = 1.0 + score * hscale
    edge = (base + delta) * mod
    out_ref[0] = jnp.where(selq > 0.0, edge, base)


def kernel(f_ir, f_vis, params):
    B, C, H, W = f_ir.shape
    N = H * W
    NP = ((N + 127) // 128) * 128
    p = params

    x2 = jnp.concatenate([f_ir, f_vis], axis=1)
    a = jnp.einsum('oc,bchw->bohw', p['agent_w1'], x2) \
        + p['agent_b1'][None, :, None, None]
    a = a / jnp.sqrt(1.0 + BN_EPS) * p['agent_bn_g'][None, :, None, None] \
        + p['agent_bn_b'][None, :, None, None]
    a = jax.nn.relu(a)
    a = jnp.einsum('oc,bchw->bohw', p['agent_w2'], a) \
        + p['agent_b2'][None, :, None, None]
    weights = jax.nn.sigmoid(a)

    imp = jnp.abs(weights - 0.5)
    gs = imp.mean(axis=(2, 3)).reshape(-1, 1)
    hh = jax.nn.relu(gs @ p['hyp_w1'].T + p['hyp_b1'])
    kr = jax.nn.sigmoid(hh @ p['hyp_w2'].T + p['hyp_b2']) * 0.8 + 0.1
    krv = kr[:, 0]
    c = krv * jnp.float32(4097.0)
    hi = c - (c - krv)
    lo = krv - hi
    prod_hi = hi * jnp.float32(N)
    prod_lo = lo * jnp.float32(N)
    fh = jnp.floor(prod_hi)
    frac = (prod_hi - fh) + prod_lo
    kf = fh + jnp.floor(frac)
    ks = jnp.maximum(kf.astype(jnp.int32), 64).reshape(B, 1, 1)

    score_flat = imp.reshape(B, 1, N)
    score_row = jnp.pad(score_flat, ((0, 0), (0, 0), (0, NP - N)),
                        constant_values=-1.0)
    w_col = jnp.pad(weights.reshape(B, N, 1),
                    ((0, 0), (0, NP - N), (0, 0)))

    xt_ir = f_ir.reshape(B, C, N).transpose(0, 2, 1)
    xt_vis = f_vis.reshape(B, C, N).transpose(0, 2, 1)
    pad = ((0, 0), (0, NP - N), (0, 0))
    xt_ir = jnp.pad(xt_ir, pad)
    xt_vis = jnp.pad(xt_vis, pad)
    xs = jnp.stack([xt_ir, xt_vis], axis=0)

    sel_row = pl.pallas_call(
        _select_body,
        grid=(B,),
        in_specs=[
            pl.BlockSpec((1, 1, 1), lambda b: (b, 0, 0),
                         memory_space=pltpu.SMEM),
            pl.BlockSpec((1, 1, NP), lambda b: (b, 0, 0)),
        ],
        out_specs=pl.BlockSpec((1, 1, NP), lambda b: (b, 0, 0)),
        out_shape=jax.ShapeDtypeStruct((B, 1, NP), jnp.float32),
    )(ks, score_row)
    sel_col = sel_row.transpose(0, 2, 1)

    def stk(name):
        return jnp.stack([p['mir' + name], p['mvis' + name]], axis=0)

    lng = stk('_ln_g').reshape(2, 1, C)
    lnb = stk('_ln_b').reshape(2, 1, C)
    inwt = jnp.stack([p['mir_in_w'].T, p['mvis_in_w'].T], axis=0)
    inb = stk('_in_b').reshape(2, 1, 3 * C)
    outwt = jnp.stack([p['mir_out_w'].T, p['mvis_out_w'].T], axis=0)
    outb = stk('_out_b').reshape(2, 1, C)
    f1t = jnp.stack([p['mir_f1_w'].T, p['mvis_f1_w'].T], axis=0)
    f1b = stk('_f1_b').reshape(2, 1, 4 * C)
    f2t = jnp.stack([p['mir_f2_w'].T, p['mvis_f2_w'].T], axis=0)
    f2b = stk('_f2_b').reshape(2, 1, C)
    hscale = p['highlight_scale'].reshape(1, 1)

    BQ = 640
    nqb = NP // BQ
    full = lambda b, q: (0, 0)
    fullm = lambda b, q: (0, 0, 0)
    out_tok = pl.pallas_call(
        _mixer_body,
        grid=(B, nqb),
        in_specs=[
            pl.BlockSpec((1, 1), full, memory_space=pltpu.SMEM),
            pl.BlockSpec((2, 1, NP, C), lambda b, q: (0, b, 0, 0)),
            pl.BlockSpec((2, 1, BQ, C), lambda b, q: (0, b, q, 0)),
            pl.BlockSpec((1, 1, NP), lambda b, q: (b, 0, 0)),
            pl.BlockSpec((1, BQ, 1), lambda b, q: (b, q, 0)),
            pl.BlockSpec((1, BQ, 1), lambda b, q: (b, q, 0)),
            pl.BlockSpec((2, 1, C), fullm),
            pl.BlockSpec((2, 1, C), fullm),
            pl.BlockSpec((2, C, 3 * C), fullm),
            pl.BlockSpec((2, 1, 3 * C), fullm),
            pl.BlockSpec((2, C, C), fullm),
            pl.BlockSpec((2, 1, C), fullm),
            pl.BlockSpec((2, C, 4 * C), fullm),
            pl.BlockSpec((2, 1, 4 * C), fullm),
            pl.BlockSpec((2, 4 * C, C), fullm),
            pl.BlockSpec((2, 1, C), fullm),
        ],
        out_specs=pl.BlockSpec((1, BQ, C), lambda b, q: (b, q, 0)),
        out_shape=jax.ShapeDtypeStruct((B, NP, C), jnp.float32),
        scratch_shapes=[
            pltpu.VMEM((2, NP, DIM), jnp.bfloat16),
            pltpu.VMEM((2, NP, 2 * DIM), jnp.bfloat16),
        ],
    )(hscale, xs, xs, sel_row, sel_col, w_col,
      lng, lnb, inwt, inb, outwt, outb, f1t, f1b, f2t, f2b)

    f_final = out_tok[:, :N, :].transpose(0, 2, 1).reshape(B, C, H, W)
    aux_loss = jnp.asarray(0.0, jnp.float32)
    return (f_final, aux_loss)
